# Initial kernel scaffold; baseline (speedup 1.0000x reference)
import jax
import jax.numpy as jnp
from jax.experimental import pallas as pl


def kernel(x1, x2, x3, xf, emb, norm_W, norm_b, xfnorm_W, xfnorm_b, qW, qb, kW, kb, vW, vb, oW, ob):
    raise NotImplementedError("write your pallas kernel here")


# speedup vs baseline: 10.4697x; 10.4697x over previous
import jax
import jax.numpy as jnp
from jax.experimental import pallas as pl

B = 2
T0, T1, T2 = 512, 512, 1024
TT = T0 + T1 + T2
N = 2048
D = 1024
XD = 1024
E = 1024
H = 16
DH = 36
TOPK = 32
NH = N // H
BT = 512

_EPS = 1e-6
_INT_MIN = -(2 ** 31)


def _mod_body(emb_ref, w_ref, b_ref, out_ref):
    e = emb_ref[...]
    se = e * jax.nn.sigmoid(e)
    y = jax.lax.dot_general(se, w_ref[...], (((1,), (1,)), ((), ())),
                            preferred_element_type=jnp.float32)
    out_ref[...] = y + b_ref[0][None, :]


def _layernorm_mod(xb, sc, sh):
    mu = jnp.mean(xb, axis=-1, keepdims=True)
    xc = xb - mu
    var = jnp.mean(xc * xc, axis=-1, keepdims=True)
    hn = xc * jax.lax.rsqrt(var + _EPS)
    return hn * (1.0 + sc)[None, :] + sh[None, :]


def _qproj_body(x_ref, mod_ref, w_ref, b_ref, out_ref):
    xb = x_ref[0]
    m = mod_ref[0, 0, 0]
    xm = _layernorm_mod(xb, m[:D], m[D:])
    y = jax.lax.dot_general(xm, w_ref[0], (((1,), (1,)), ((), ())),
                            preferred_element_type=jnp.float32)
    out_ref[0] = y + b_ref[0, 0][None, :]


def _kvproj_body(xf_ref, mod_ref, kw_ref, kb_ref, vw_ref, vb_ref, k_ref, v_ref):
    xb = xf_ref[0]
    m = mod_ref[0, 0, 0]
    xm = _layernorm_mod(xb, m[:XD], m[XD:])
    k = jax.lax.dot_general(xm, kw_ref[0], (((1,), (1,)), ((), ())),
                            preferred_element_type=jnp.float32)
    v = jax.lax.dot_general(xm, vw_ref[0], (((1,), (1,)), ((), ())),
                            preferred_element_type=jnp.float32)
    k_ref[0, 0] = k + kb_ref[0, 0][None, :]
    v_ref[0, 0] = v + vb_ref[0, 0][None, :]


def _attn_body(q_ref, k_ref, v_ref, out_ref):
    q = q_ref[0, 0]
    k = k_ref[0, 0]
    v = v_ref[0, 0]
    l = jax.lax.dot_general(q, k, (((1,), (1,)), ((), ())),
                            preferred_element_type=jnp.float32) * (1.0 / 6.0)
    m = jnp.max(l, axis=-1, keepdims=True)
    e = jnp.exp(l - m)

    bits = jax.lax.bitcast_convert_type(l, jnp.int32)
    key = jnp.where(bits < 0, bits ^ 0x7FFFFFFF, bits)
    t = jnp.full((l.shape[0], 1), _INT_MIN, jnp.int32)
    cnt = jnp.sum((key >= 0).astype(jnp.int32), axis=-1, keepdims=True)
    t = jnp.where(cnt >= TOPK, jnp.zeros_like(t), t)
    for b in range(30, -1, -1):
        cand = t | (1 << b)
        cnt = jnp.sum((key >= cand).astype(jnp.int32), axis=-1, keepdims=True)
        t = jnp.where(cnt >= TOPK, cand, t)

    w = jnp.where(key >= t, e, 0.0)
    s = jnp.sum(w, axis=-1, keepdims=True)
    o = jax.lax.dot_general(w, v, (((1,), (0,)), ((), ())),
                            preferred_element_type=jnp.float32)
    out_ref[0, 0] = o / s


def _oproj_body(x_ref, w_ref, b_ref, out_ref):
    y = jax.lax.dot_general(x_ref[0], w_ref[0], (((1,), (1,)), ((), ())),
                            preferred_element_type=jnp.float32)
    out_ref[0] = y + b_ref[0, 0][None, :]


def kernel(x1, x2, x3, xf, emb, norm_W, norm_b, xfnorm_W, xfnorm_b, qW, qb, kW, kb, vW, vb, oW, ob):
    f32 = jnp.float32

    Wall = jnp.concatenate([norm_W.reshape(3 * 2 * D, E), xfnorm_W], axis=0)
    ball = jnp.concatenate([norm_b.reshape(3 * 2 * D), xfnorm_b], axis=0).reshape(1, -1)
    MODCOLS = Wall.shape[0]
    MB = 1024
    mod = pl.pallas_call(
        _mod_body,
        grid=(MODCOLS // MB,),
        in_specs=[
            pl.BlockSpec((B, E), lambda j: (0, 0)),
            pl.BlockSpec((MB, E), lambda j: (j, 0)),
            pl.BlockSpec((1, MB), lambda j: (0, j)),
        ],
        out_specs=pl.BlockSpec((B, MB), lambda j: (0, j)),
        out_shape=jax.ShapeDtypeStruct((B, MODCOLS), f32),
    )(emb, Wall, ball)
    mod_q = mod[:, :3 * 2 * D].reshape(B, 3, 1, 2 * D)
    mod_f = mod[:, 3 * 2 * D:].reshape(B, 1, 1, 2 * XD)

    x = jnp.concatenate([x1, x2, x3], axis=1)
    qb3 = qb.reshape(3, 1, H * DH)
    nblk = TT // BT

    def sid(j):
        return jnp.minimum(j, 2)

    Q = pl.pallas_call(
        _qproj_body,
        grid=(B, nblk),
        in_specs=[
            pl.BlockSpec((1, BT, D), lambda b, j: (b, j, 0)),
            pl.BlockSpec((1, 1, 1, 2 * D), lambda b, j: (b, sid(j), 0, 0)),
            pl.BlockSpec((1, H * DH, D), lambda b, j: (sid(j), 0, 0)),
            pl.BlockSpec((1, 1, H * DH), lambda b, j: (sid(j), 0, 0)),
        ],
        out_specs=pl.BlockSpec((1, BT, H * DH), lambda b, j: (b, j, 0)),
        out_shape=jax.ShapeDtypeStruct((B, TT, H * DH), f32),
    )(x, mod_q, qW, qb3)

    kb3 = kb.reshape(H, 1, DH)
    vb3 = vb.reshape(H, 1, DH)
    K, V = pl.pallas_call(
        _kvproj_body,
        grid=(B, H),
        in_specs=[
            pl.BlockSpec((1, NH, XD), lambda b, h: (b, h, 0)),
            pl.BlockSpec((1, 1, 1, 2 * XD), lambda b, h: (b, 0, 0, 0)),
            pl.BlockSpec((1, DH, XD), lambda b, h: (h, 0, 0)),
            pl.BlockSpec((1, 1, DH), lambda b, h: (h, 0, 0)),
            pl.BlockSpec((1, DH, XD), lambda b, h: (h, 0, 0)),
            pl.BlockSpec((1, 1, DH), lambda b, h: (h, 0, 0)),
        ],
        out_specs=[
            pl.BlockSpec((1, 1, NH, DH), lambda b, h: (b, h, 0, 0)),
            pl.BlockSpec((1, 1, NH, DH), lambda b, h: (b, h, 0, 0)),
        ],
        out_shape=[
            jax.ShapeDtypeStruct((B, H, NH, DH), f32),
            jax.ShapeDtypeStruct((B, H, NH, DH), f32),
        ],
    )(xf, mod_f, kW, kb3, vW, vb3)

    Qh = Q.reshape(B, TT, H, DH).transpose(0, 2, 1, 3)
    O = pl.pallas_call(
        _attn_body,
        grid=(B, H, nblk),
        in_specs=[
            pl.BlockSpec((1, 1, BT, DH), lambda b, h, j: (b, h, j, 0)),
            pl.BlockSpec((1, 1, NH, DH), lambda b, h, j: (b, h, 0, 0)),
            pl.BlockSpec((1, 1, NH, DH), lambda b, h, j: (b, h, 0, 0)),
        ],
        out_specs=pl.BlockSpec((1, 1, BT, DH), lambda b, h, j: (b, h, j, 0)),
        out_shape=jax.ShapeDtypeStruct((B, H, TT, DH), f32),
    )(Qh, K, V)
    out = O.transpose(0, 2, 1, 3).reshape(B, TT, H * DH)

    ob3 = ob.reshape(3, 1, D)
    y = pl.pallas_call(
        _oproj_body,
        grid=(B, nblk),
        in_specs=[
            pl.BlockSpec((1, BT, H * DH), lambda b, j: (b, j, 0)),
            pl.BlockSpec((1, D, H * DH), lambda b, j: (sid(j), 0, 0)),
            pl.BlockSpec((1, 1, D), lambda b, j: (sid(j), 0, 0)),
        ],
        out_specs=pl.BlockSpec((1, BT, D), lambda b, j: (b, j, 0)),
        out_shape=jax.ShapeDtypeStruct((B, TT, D), f32),
    )(out, oW, ob3)

    return (y[:, :T0], y[:, T0:T0 + T1], y[:, T0 + T1:])


# speedup vs baseline: 21.9638x; 2.0979x over previous
import jax
import jax.numpy as jnp
from jax.experimental import pallas as pl

B = 2
T0, T1, T2 = 512, 512, 1024
TT = T0 + T1 + T2
N = 2048
D = 1024
XD = 1024
E = 1024
H = 16
DH = 36
TOPK = 32
NH = N // H
BT = 512

_EPS = 1e-6
_INT_MIN = -(2 ** 31)


def _mod_body(emb_ref, w_ref, b_ref, out_ref):
    e = emb_ref[...]
    se = e * jax.nn.sigmoid(e)
    y = jax.lax.dot_general(se, w_ref[...], (((1,), (1,)), ((), ())),
                            preferred_element_type=jnp.float32)
    out_ref[...] = y + b_ref[0][None, :]


def _layernorm_mod(xb, sc, sh):
    mu = jnp.mean(xb, axis=-1, keepdims=True)
    xc = xb - mu
    var = jnp.mean(xc * xc, axis=-1, keepdims=True)
    hn = xc * jax.lax.rsqrt(var + _EPS)
    return hn * (1.0 + sc)[None, :] + sh[None, :]


def _qproj_body(x_ref, mod_ref, w_ref, b_ref, out_ref):
    xb = x_ref[0]
    m = mod_ref[0, 0, 0]
    xm = _layernorm_mod(xb, m[:D], m[D:])
    y = jax.lax.dot_general(xm, w_ref[0], (((1,), (1,)), ((), ())),
                            preferred_element_type=jnp.float32)
    out_ref[0] = y + b_ref[0, 0][None, :]


def _kvproj_body(xf_ref, mod_ref, kw_ref, kb_ref, vw_ref, vb_ref, k_ref, v_ref):
    xb = xf_ref[0]
    m = mod_ref[0, 0, 0]
    xm = _layernorm_mod(xb, m[:XD], m[XD:])
    k = jax.lax.dot_general(xm, kw_ref[0], (((1,), (1,)), ((), ())),
                            preferred_element_type=jnp.float32)
    v = jax.lax.dot_general(xm, vw_ref[0], (((1,), (1,)), ((), ())),
                            preferred_element_type=jnp.float32)
    k_ref[0, 0] = k + kb_ref[0, 0][None, :]
    v_ref[0, 0] = v + vb_ref[0, 0][None, :]


def _attn_body(q_ref, k_ref, v_ref, out_ref):
    q = q_ref[0, 0]
    k = k_ref[0, 0]
    v = v_ref[0, 0]
    lT = jax.lax.dot_general(k, q, (((1,), (1,)), ((), ())),
                             preferred_element_type=jnp.float32) * (1.0 / 6.0)
    m = jnp.max(lT, axis=0, keepdims=True)
    eT = jnp.exp(lT - m)

    bits = jax.lax.bitcast_convert_type(lT, jnp.int32)
    key = jnp.where(bits < 0, bits ^ 0x7FFFFFFF, bits)
    t = jnp.full((1, lT.shape[1]), _INT_MIN, jnp.int32)
    cnt = jnp.sum((key >= 0).astype(jnp.int32), axis=0, keepdims=True)
    t = jnp.where(cnt >= TOPK, jnp.zeros_like(t), t)
    for b in range(30, 7, -1):
        cand = t | (1 << b)
        cnt = jnp.sum((key >= cand).astype(jnp.int32), axis=0, keepdims=True)
        t = jnp.where(cnt >= TOPK, cand, t)

    wT = jnp.where(key >= t, eT, 0.0)
    s = jnp.sum(wT, axis=0, keepdims=True)
    wn = wT * (1.0 / s)
    o = jax.lax.dot_general(wn, v, (((0,), (0,)), ((), ())),
                            preferred_element_type=jnp.float32)
    out_ref[0, 0] = o


def _oproj_body(x_ref, w_ref, b_ref, out_ref):
    y = jax.lax.dot_general(x_ref[0], w_ref[0], (((1,), (1,)), ((), ())),
                            preferred_element_type=jnp.float32)
    out_ref[0] = y + b_ref[0, 0][None, :]


def kernel(x1, x2, x3, xf, emb, norm_W, norm_b, xfnorm_W, xfnorm_b, qW, qb, kW, kb, vW, vb, oW, ob):
    f32 = jnp.float32

    Wall = jnp.concatenate([norm_W.reshape(3 * 2 * D, E), xfnorm_W], axis=0)
    ball = jnp.concatenate([norm_b.reshape(3 * 2 * D), xfnorm_b], axis=0).reshape(1, -1)
    MODCOLS = Wall.shape[0]
    MB = 1024
    mod = pl.pallas_call(
        _mod_body,
        grid=(MODCOLS // MB,),
        in_specs=[
            pl.BlockSpec((B, E), lambda j: (0, 0)),
            pl.BlockSpec((MB, E), lambda j: (j, 0)),
            pl.BlockSpec((1, MB), lambda j: (0, j)),
        ],
        out_specs=pl.BlockSpec((B, MB), lambda j: (0, j)),
        out_shape=jax.ShapeDtypeStruct((B, MODCOLS), f32),
    )(emb, Wall, ball)
    mod_q = mod[:, :3 * 2 * D].reshape(B, 3, 1, 2 * D)
    mod_f = mod[:, 3 * 2 * D:].reshape(B, 1, 1, 2 * XD)

    x = jnp.concatenate([x1, x2, x3], axis=1)
    qb3 = qb.reshape(3, 1, H * DH)
    nblk = TT // BT

    def sid(j):
        return jnp.minimum(j, 2)

    Q = pl.pallas_call(
        _qproj_body,
        grid=(B, nblk),
        in_specs=[
            pl.BlockSpec((1, BT, D), lambda b, j: (b, j, 0)),
            pl.BlockSpec((1, 1, 1, 2 * D), lambda b, j: (b, sid(j), 0, 0)),
            pl.BlockSpec((1, H * DH, D), lambda b, j: (sid(j), 0, 0)),
            pl.BlockSpec((1, 1, H * DH), lambda b, j: (sid(j), 0, 0)),
        ],
        out_specs=pl.BlockSpec((1, BT, H * DH), lambda b, j: (b, j, 0)),
        out_shape=jax.ShapeDtypeStruct((B, TT, H * DH), f32),
    )(x, mod_q, qW, qb3)

    kb3 = kb.reshape(H, 1, DH)
    vb3 = vb.reshape(H, 1, DH)
    K, V = pl.pallas_call(
        _kvproj_body,
        grid=(B, H),
        in_specs=[
            pl.BlockSpec((1, NH, XD), lambda b, h: (b, h, 0)),
            pl.BlockSpec((1, 1, 1, 2 * XD), lambda b, h: (b, 0, 0, 0)),
            pl.BlockSpec((1, DH, XD), lambda b, h: (h, 0, 0)),
            pl.BlockSpec((1, 1, DH), lambda b, h: (h, 0, 0)),
            pl.BlockSpec((1, DH, XD), lambda b, h: (h, 0, 0)),
            pl.BlockSpec((1, 1, DH), lambda b, h: (h, 0, 0)),
        ],
        out_specs=[
            pl.BlockSpec((1, 1, NH, DH), lambda b, h: (b, h, 0, 0)),
            pl.BlockSpec((1, 1, NH, DH), lambda b, h: (b, h, 0, 0)),
        ],
        out_shape=[
            jax.ShapeDtypeStruct((B, H, NH, DH), f32),
            jax.ShapeDtypeStruct((B, H, NH, DH), f32),
        ],
    )(xf, mod_f, kW, kb3, vW, vb3)

    Qh = Q.reshape(B, TT, H, DH).transpose(0, 2, 1, 3)
    O = pl.pallas_call(
        _attn_body,
        grid=(B, H, nblk),
        in_specs=[
            pl.BlockSpec((1, 1, BT, DH), lambda b, h, j: (b, h, j, 0)),
            pl.BlockSpec((1, 1, NH, DH), lambda b, h, j: (b, h, 0, 0)),
            pl.BlockSpec((1, 1, NH, DH), lambda b, h, j: (b, h, 0, 0)),
        ],
        out_specs=pl.BlockSpec((1, 1, BT, DH), lambda b, h, j: (b, h, j, 0)),
        out_shape=jax.ShapeDtypeStruct((B, H, TT, DH), f32),
    )(Qh, K, V)
    out = O.transpose(0, 2, 1, 3).reshape(B, TT, H * DH)

    ob3 = ob.reshape(3, 1, D)
    y = pl.pallas_call(
        _oproj_body,
        grid=(B, nblk),
        in_specs=[
            pl.BlockSpec((1, BT, H * DH), lambda b, j: (b, j, 0)),
            pl.BlockSpec((1, D, H * DH), lambda b, j: (sid(j), 0, 0)),
            pl.BlockSpec((1, 1, D), lambda b, j: (sid(j), 0, 0)),
        ],
        out_specs=pl.BlockSpec((1, BT, D), lambda b, j: (b, j, 0)),
        out_shape=jax.ShapeDtypeStruct((B, TT, D), f32),
    )(out, oW, ob3)

    return (y[:, :T0], y[:, T0:T0 + T1], y[:, T0 + T1:])


# speedup vs baseline: 34.3898x; 1.5657x over previous
import jax
import jax.numpy as jnp
from jax.experimental import pallas as pl

B = 2
T0, T1, T2 = 512, 512, 1024
TT = T0 + T1 + T2
N = 2048
D = 1024
XD = 1024
E = 1024
H = 16
DH = 36
TOPK = 32
NH = N // H
BT = 512

_EPS = 1e-6
_INT_MIN = -(2 ** 31)


def _mod_body(emb_ref, w_ref, b_ref, out_ref):
    e = emb_ref[...]
    se = e * jax.nn.sigmoid(e)
    y = jax.lax.dot_general(se, w_ref[...], (((1,), (1,)), ((), ())),
                            preferred_element_type=jnp.float32)
    out_ref[...] = y + b_ref[0][None, :]


def _layernorm_mod(xb, sc, sh):
    mu = jnp.mean(xb, axis=-1, keepdims=True)
    xc = xb - mu
    var = jnp.mean(xc * xc, axis=-1, keepdims=True)
    hn = xc * jax.lax.rsqrt(var + _EPS)
    return hn * (1.0 + sc)[None, :] + sh[None, :]


def _qproj_body(x_ref, mod_ref, w_ref, b_ref, out_ref):
    xb = x_ref[0]
    m = mod_ref[0, 0, 0]
    xm = _layernorm_mod(xb, m[:D], m[D:])
    y = jax.lax.dot_general(xm, w_ref[0], (((1,), (1,)), ((), ())),
                            preferred_element_type=jnp.float32)
    y = y + b_ref[0, 0][None, :]
    for h in range(H):
        out_ref[0, h] = y[:, h * DH:(h + 1) * DH]


def _kvproj_body(xf_ref, mod_ref, kw_ref, kb_ref, vw_ref, vb_ref, k_ref, v_ref):
    xb = xf_ref[0]
    m = mod_ref[0, 0, 0]
    xm = _layernorm_mod(xb, m[:XD], m[XD:])
    k = jax.lax.dot_general(xm, kw_ref[0], (((1,), (1,)), ((), ())),
                            preferred_element_type=jnp.float32)
    v = jax.lax.dot_general(xm, vw_ref[0], (((1,), (1,)), ((), ())),
                            preferred_element_type=jnp.float32)
    k_ref[0, 0] = k + kb_ref[0, 0][None, :]
    v_ref[0, 0] = v + vb_ref[0, 0][None, :]


def _head_attn(qh, kh, vh):
    lT = jax.lax.dot_general(kh, qh, (((1,), (1,)), ((), ())),
                             preferred_element_type=jnp.float32) * (1.0 / 6.0)
    m = jnp.max(lT, axis=0, keepdims=True)
    eT = jnp.exp(lT - m)

    bits = jax.lax.bitcast_convert_type(lT, jnp.int32)
    key = jnp.where(bits < 0, bits ^ 0x7FFFFFFF, bits)
    t = jnp.full((1, lT.shape[1]), _INT_MIN, jnp.int32)
    cnt = jnp.sum((key >= 0).astype(jnp.int32), axis=0, keepdims=True)
    t = jnp.where(cnt >= TOPK, jnp.zeros_like(t), t)
    for b in range(30, 7, -1):
        cand = t | (1 << b)
        cnt = jnp.sum((key >= cand).astype(jnp.int32), axis=0, keepdims=True)
        t = jnp.where(cnt >= TOPK, cand, t)

    wT = jnp.where(key >= t, eT, 0.0)
    s = jnp.sum(wT, axis=0, keepdims=True)
    wn = wT * (1.0 / s)
    return jax.lax.dot_general(wn, vh, (((0,), (0,)), ((), ())),
                               preferred_element_type=jnp.float32)


def _attnproj_body(q_ref, k_ref, v_ref, ow_ref, ob_ref, out_ref):
    os = [_head_attn(q_ref[0, h], k_ref[0, h], v_ref[0, h]) for h in range(H)]
    o = jnp.concatenate(os, axis=1)
    z = jax.lax.dot_general(o, ow_ref[0], (((1,), (1,)), ((), ())),
                            preferred_element_type=jnp.float32)
    out_ref[0] = z + ob_ref[0, 0][None, :]


def kernel(x1, x2, x3, xf, emb, norm_W, norm_b, xfnorm_W, xfnorm_b, qW, qb, kW, kb, vW, vb, oW, ob):
    f32 = jnp.float32
    nblk = TT // BT

    def sid(j):
        return jnp.minimum(j, 2)

    MB = 1024

    def _mod_call(W2, b2):
        cols = W2.shape[0]
        return pl.pallas_call(
            _mod_body,
            grid=(cols // MB,),
            in_specs=[
                pl.BlockSpec((B, E), lambda j: (0, 0)),
                pl.BlockSpec((MB, E), lambda j: (j, 0)),
                pl.BlockSpec((1, MB), lambda j: (0, j)),
            ],
            out_specs=pl.BlockSpec((B, MB), lambda j: (0, j)),
            out_shape=jax.ShapeDtypeStruct((B, cols), f32),
        )(emb, W2, b2)

    mod_q = _mod_call(norm_W.reshape(3 * 2 * D, E),
                      norm_b.reshape(1, 3 * 2 * D)).reshape(B, 3, 1, 2 * D)
    mod_f = _mod_call(xfnorm_W, xfnorm_b.reshape(1, 2 * XD)).reshape(B, 1, 1, 2 * XD)

    x = jnp.concatenate([x1, x2, x3], axis=1)
    qb3 = qb.reshape(3, 1, H * DH)
    Q = pl.pallas_call(
        _qproj_body,
        grid=(nblk, B),
        in_specs=[
            pl.BlockSpec((1, BT, D), lambda j, b: (b, j, 0)),
            pl.BlockSpec((1, 1, 1, 2 * D), lambda j, b: (b, sid(j), 0, 0)),
            pl.BlockSpec((1, H * DH, D), lambda j, b: (sid(j), 0, 0)),
            pl.BlockSpec((1, 1, H * DH), lambda j, b: (sid(j), 0, 0)),
        ],
        out_specs=pl.BlockSpec((1, H, BT, DH), lambda j, b: (b, 0, j, 0)),
        out_shape=jax.ShapeDtypeStruct((B, H, TT, DH), f32),
    )(x, mod_q, qW, qb3)

    kb3 = kb.reshape(H, 1, DH)
    vb3 = vb.reshape(H, 1, DH)
    K, V = pl.pallas_call(
        _kvproj_body,
        grid=(B, H),
        in_specs=[
            pl.BlockSpec((1, NH, XD), lambda b, h: (b, h, 0)),
            pl.BlockSpec((1, 1, 1, 2 * XD), lambda b, h: (b, 0, 0, 0)),
            pl.BlockSpec((1, DH, XD), lambda b, h: (h, 0, 0)),
            pl.BlockSpec((1, 1, DH), lambda b, h: (h, 0, 0)),
            pl.BlockSpec((1, DH, XD), lambda b, h: (h, 0, 0)),
            pl.BlockSpec((1, 1, DH), lambda b, h: (h, 0, 0)),
        ],
        out_specs=[
            pl.BlockSpec((1, 1, NH, DH), lambda b, h: (b, h, 0, 0)),
            pl.BlockSpec((1, 1, NH, DH), lambda b, h: (b, h, 0, 0)),
        ],
        out_shape=[
            jax.ShapeDtypeStruct((B, H, NH, DH), f32),
            jax.ShapeDtypeStruct((B, H, NH, DH), f32),
        ],
    )(xf, mod_f, kW, kb3, vW, vb3)

    ob3 = ob.reshape(3, 1, D)
    y = pl.pallas_call(
        _attnproj_body,
        grid=(nblk, B),
        in_specs=[
            pl.BlockSpec((1, H, BT, DH), lambda j, b: (b, 0, j, 0)),
            pl.BlockSpec((1, H, NH, DH), lambda j, b: (b, 0, 0, 0)),
            pl.BlockSpec((1, H, NH, DH), lambda j, b: (b, 0, 0, 0)),
            pl.BlockSpec((1, D, H * DH), lambda j, b: (sid(j), 0, 0)),
            pl.BlockSpec((1, 1, D), lambda j, b: (sid(j), 0, 0)),
        ],
        out_specs=pl.BlockSpec((1, BT, D), lambda j, b: (b, j, 0)),
        out_shape=jax.ShapeDtypeStruct((B, TT, D), f32),
    )(Q, K, V, oW, ob3)

    return (y[:, :T0], y[:, T0:T0 + T1], y[:, T0 + T1:])


# speedup vs baseline: 38.0245x; 1.1057x over previous
import jax
import jax.numpy as jnp
from jax.experimental import pallas as pl

B = 2
T0, T1, T2 = 512, 512, 1024
TT = T0 + T1 + T2
N = 2048
D = 1024
XD = 1024
E = 1024
H = 16
DH = 36
TOPK = 32
NH = N // H
BT = 512

_EPS = 1e-6
_INT_MIN = -(2 ** 31)


def _mod_body(emb_ref, w1_ref, b1_ref, w2_ref, b2_ref, mq_ref, mf_ref):
    e = emb_ref[...]
    se = e * jax.nn.sigmoid(e)
    y1 = jax.lax.dot_general(se, w1_ref[...], (((1,), (1,)), ((), ())),
                             preferred_element_type=jnp.float32)
    mq_ref[...] = y1 + b1_ref[0][None, :]
    y2 = jax.lax.dot_general(se, w2_ref[...], (((1,), (1,)), ((), ())),
                             preferred_element_type=jnp.float32)
    mf_ref[...] = y2 + b2_ref[0][None, :]


def _layernorm_mod(xb, sc, sh):
    mu = jnp.mean(xb, axis=-1, keepdims=True)
    xc = xb - mu
    var = jnp.mean(xc * xc, axis=-1, keepdims=True)
    hn = xc * jax.lax.rsqrt(var + _EPS)
    return hn * (1.0 + sc)[None, :] + sh[None, :]


def _kvproj_body(xf_ref, mod_ref, kw_ref, kb_ref, vw_ref, vb_ref, k_ref, v_ref):
    xb = xf_ref[0]
    m = mod_ref[0, 0, 0]
    xm = _layernorm_mod(xb, m[:XD], m[XD:])
    k = jax.lax.dot_general(xm, kw_ref[0], (((1,), (1,)), ((), ())),
                            preferred_element_type=jnp.float32)
    v = jax.lax.dot_general(xm, vw_ref[0], (((1,), (1,)), ((), ())),
                            preferred_element_type=jnp.float32)
    k_ref[0, 0] = k + kb_ref[0, 0][None, :]
    v_ref[0, 0] = v + vb_ref[0, 0][None, :]


def _head_attn(qh, kh, vh):
    lT = jax.lax.dot_general(kh, qh, (((1,), (1,)), ((), ())),
                             preferred_element_type=jnp.float32) * (1.0 / 6.0)
    m = jnp.max(lT, axis=0, keepdims=True)
    eT = jnp.exp(lT - m)

    bits = jax.lax.bitcast_convert_type(lT, jnp.int32)
    key = jnp.where(bits < 0, bits ^ 0x7FFFFFFF, bits)
    t = jnp.full((1, lT.shape[1]), _INT_MIN, jnp.int32)
    cnt = jnp.sum((key >= 0).astype(jnp.int32), axis=0, keepdims=True)
    t = jnp.where(cnt >= TOPK, jnp.zeros_like(t), t)
    for b in range(30, 9, -1):
        cand = t | (1 << b)
        cnt = jnp.sum((key >= cand).astype(jnp.int32), axis=0, keepdims=True)
        t = jnp.where(cnt >= TOPK, cand, t)

    wT = jnp.where(key >= t, eT, 0.0)
    s = jnp.sum(wT, axis=0, keepdims=True)
    wn = wT * (1.0 / s)
    return jax.lax.dot_general(wn, vh, (((0,), (0,)), ((), ())),
                               preferred_element_type=jnp.float32)


def _mega_body(x1_ref, x2_ref, x3_ref, mod_ref, qw_ref, qb_ref,
               k_ref, v_ref, ow_ref, ob_ref, out_ref):
    j = pl.program_id(0)
    sidv = jnp.minimum(j, 2)
    xb = jnp.where(sidv == 0, x1_ref[0],
                   jnp.where(sidv == 1, x2_ref[0], x3_ref[0]))
    m = mod_ref[0, 0, 0]
    xm = _layernorm_mod(xb, m[:D], m[D:])
    y = jax.lax.dot_general(xm, qw_ref[0], (((1,), (1,)), ((), ())),
                            preferred_element_type=jnp.float32)
    y = y + qb_ref[0, 0][None, :]
    os = [_head_attn(y[:, h * DH:(h + 1) * DH], k_ref[0, h], v_ref[0, h])
          for h in range(H)]
    o = jnp.concatenate(os, axis=1)
    z = jax.lax.dot_general(o, ow_ref[0], (((1,), (1,)), ((), ())),
                            preferred_element_type=jnp.float32)
    out_ref[0] = z + ob_ref[0, 0][None, :]


def kernel(x1, x2, x3, xf, emb, norm_W, norm_b, xfnorm_W, xfnorm_b, qW, qb, kW, kb, vW, vb, oW, ob):
    f32 = jnp.float32
    nblk = TT // BT

    def sid(j):
        return jnp.minimum(j, 2)

    MQ = 3 * 2 * D
    MF = 2 * XD
    mq, mf = pl.pallas_call(
        _mod_body,
        grid=(2,),
        in_specs=[
            pl.BlockSpec((B, E), lambda j: (0, 0)),
            pl.BlockSpec((MQ // 2, E), lambda j: (j, 0)),
            pl.BlockSpec((1, MQ // 2), lambda j: (0, j)),
            pl.BlockSpec((MF // 2, E), lambda j: (j, 0)),
            pl.BlockSpec((1, MF // 2), lambda j: (0, j)),
        ],
        out_specs=[
            pl.BlockSpec((B, MQ // 2), lambda j: (0, j)),
            pl.BlockSpec((B, MF // 2), lambda j: (0, j)),
        ],
        out_shape=[
            jax.ShapeDtypeStruct((B, MQ), f32),
            jax.ShapeDtypeStruct((B, MF), f32),
        ],
    )(emb, norm_W.reshape(MQ, E), norm_b.reshape(1, MQ),
      xfnorm_W, xfnorm_b.reshape(1, MF))
    mod_q = mq.reshape(B, 3, 1, 2 * D)
    mod_f = mf.reshape(B, 1, 1, 2 * XD)

    kb3 = kb.reshape(H, 1, DH)
    vb3 = vb.reshape(H, 1, DH)
    K, V = pl.pallas_call(
        _kvproj_body,
        grid=(B, H),
        in_specs=[
            pl.BlockSpec((1, NH, XD), lambda b, h: (b, h, 0)),
            pl.BlockSpec((1, 1, 1, 2 * XD), lambda b, h: (b, 0, 0, 0)),
            pl.BlockSpec((1, DH, XD), lambda b, h: (h, 0, 0)),
            pl.BlockSpec((1, 1, DH), lambda b, h: (h, 0, 0)),
            pl.BlockSpec((1, DH, XD), lambda b, h: (h, 0, 0)),
            pl.BlockSpec((1, 1, DH), lambda b, h: (h, 0, 0)),
        ],
        out_specs=[
            pl.BlockSpec((1, 1, NH, DH), lambda b, h: (b, h, 0, 0)),
            pl.BlockSpec((1, 1, NH, DH), lambda b, h: (b, h, 0, 0)),
        ],
        out_shape=[
            jax.ShapeDtypeStruct((B, H, NH, DH), f32),
            jax.ShapeDtypeStruct((B, H, NH, DH), f32),
        ],
    )(xf, mod_f, kW, kb3, vW, vb3)

    qb3 = qb.reshape(3, 1, H * DH)
    ob3 = ob.reshape(3, 1, D)
    y = pl.pallas_call(
        _mega_body,
        grid=(nblk, B),
        in_specs=[
            pl.BlockSpec((1, BT, D), lambda j, b: (b, 0, 0)),
            pl.BlockSpec((1, BT, D), lambda j, b: (b, 0, 0)),
            pl.BlockSpec((1, BT, D), lambda j, b: (b, jnp.maximum(j - 2, 0), 0)),
            pl.BlockSpec((1, 1, 1, 2 * D), lambda j, b: (b, sid(j), 0, 0)),
            pl.BlockSpec((1, H * DH, D), lambda j, b: (sid(j), 0, 0)),
            pl.BlockSpec((1, 1, H * DH), lambda j, b: (sid(j), 0, 0)),
            pl.BlockSpec((1, H, NH, DH), lambda j, b: (b, 0, 0, 0)),
            pl.BlockSpec((1, H, NH, DH), lambda j, b: (b, 0, 0, 0)),
            pl.BlockSpec((1, D, H * DH), lambda j, b: (sid(j), 0, 0)),
            pl.BlockSpec((1, 1, D), lambda j, b: (sid(j), 0, 0)),
        ],
        out_specs=pl.BlockSpec((1, BT, D), lambda j, b: (b, j, 0)),
        out_shape=jax.ShapeDtypeStruct((B, TT, D), f32),
    )(x1, x2, x3, mod_q, qW, qb3, K, V, oW, ob3)

    return (y[:, :T0], y[:, T0:T0 + T1], y[:, T0 + T1:])
